# Initial kernel scaffold; baseline (speedup 1.0000x reference)
import jax
import jax.numpy as jnp
from jax.experimental import pallas as pl


def kernel(nlist, positions, box, sample_weight, mol_indices):
    raise NotImplementedError("write your pallas kernel here")


# speedup vs baseline: 1.4047x; 1.4047x over previous
import functools

import jax
import jax.numpy as jnp
import numpy as np
from jax import lax
from jax.experimental import pallas as pl
from jax.experimental.pallas import tpu as pltpu
from jax.experimental.pallas import tpu_sc as plsc

_N_ATOMS = 32768
_MN = 8
_N_MOL = _N_ATOMS // _MN
_NN = 64
_F = 4 * _NN

_BLK = 512
_NBLK = _N_ATOMS // _BLK

_NC, _NS = 2, 16
_NW = _NC * _NS
_MPW = _N_MOL // _NW
_IPW = _MPW * _MN

_S_NP = np.zeros((_F, _NN), np.float32)
for _i in range(_F):
    if _i % 4 < 3:
        _S_NP[_i, _i // 4] = 1.0


def _atom_energy_body(n2_ref, s_ref, sw_ref, out_ref):
    x = n2_ref[...]
    sq = x * x
    r2 = jnp.dot(sq, s_ref[...], precision=lax.Precision.HIGHEST,
                 preferred_element_type=jnp.float32)
    pred = r2 > 1e-6
    r2s = jnp.where(pred, r2, 1.0)
    r6 = r2s * r2s * r2s
    inv6 = 1.0 / r6
    scale = 2.0 * sw_ref[0, 0]
    pe = jnp.where(pred, scale * (inv6 * inv6 - inv6), 0.0)
    out_ref[0, 0, :] = jnp.sum(pe, axis=1)


def _atom_energies(n2, s_mat, sw):
    return pl.pallas_call(
        _atom_energy_body,
        grid=(_NBLK,),
        in_specs=[
            pl.BlockSpec((_BLK, _F), lambda i: (i, 0)),
            pl.BlockSpec((_F, _NN), lambda i: (0, 0)),
            pl.BlockSpec(memory_space=pltpu.SMEM),
        ],
        out_specs=pl.BlockSpec((1, 1, _BLK), lambda i: (i, 0, 0)),
        out_shape=jax.ShapeDtypeStruct((_NBLK, 1, _BLK), jnp.float32),
    )(n2, s_mat, sw)


def _mol_sum_body(idx_hbm, epad_hbm, out_hbm, idx_v, rows_v, out_v, sem):
    wid = lax.axis_index("s") * _NC + lax.axis_index("c")
    pltpu.sync_copy(idx_hbm.at[pl.ds(wid * _IPW, _IPW)], idx_v)
    for s in range(_MN):
        pltpu.async_copy(
            epad_hbm.at[idx_v.at[pl.ds(s * _MPW, _MPW)]],
            rows_v.at[pl.ds(s * _MPW, _MPW)], sem).wait()
    for k in range(_MPW // 16):
        acc = rows_v[pl.ds(k * 16, 16)]
        for s in range(1, _MN):
            acc = acc + rows_v[pl.ds(s * _MPW + k * 16, 16)]
        out_v[pl.ds(k * 16, 16)] = acc
    pltpu.sync_copy(out_v, out_hbm.at[pl.ds(wid * _MPW, _MPW)])


def _mol_sum(idx_t, e_pad):
    mesh = plsc.VectorSubcoreMesh(core_axis_name="c", subcore_axis_name="s")
    fn = functools.partial(
        pl.kernel,
        out_type=jax.ShapeDtypeStruct((_N_MOL,), jnp.float32),
        mesh=mesh,
        scratch_types=[
            pltpu.VMEM((_IPW,), jnp.int32),
            pltpu.VMEM((_IPW,), jnp.float32),
            pltpu.VMEM((_MPW,), jnp.float32),
            pltpu.SemaphoreType.DMA,
        ],
    )(_mol_sum_body)
    return fn(idx_t, e_pad)


def kernel(nlist, positions, box, sample_weight, mol_indices):
    n2 = nlist.reshape(_N_ATOMS, _F)
    sw = jnp.reshape(sample_weight, (1, 1)).astype(jnp.float32)
    e = _atom_energies(n2, jnp.asarray(_S_NP), sw).reshape(_N_ATOMS)
    e_pad = jnp.pad(e, (1, 7))
    idx_t = (mol_indices.reshape(_NW, _MPW, _MN)
             .transpose(0, 2, 1).reshape(_NW * _IPW))
    return _mol_sum(idx_t, e_pad)


# speedup vs baseline: 1.5059x; 1.0720x over previous
import functools

import jax
import jax.numpy as jnp
import numpy as np
from jax import lax
from jax.experimental import pallas as pl
from jax.experimental.pallas import tpu as pltpu
from jax.experimental.pallas import tpu_sc as plsc

_N_ATOMS = 32768
_MN = 8
_N_MOL = _N_ATOMS // _MN
_NN = 64
_F = 4 * _NN

_BLK = 512
_NBLK = _N_ATOMS // _BLK

_NC, _NS = 2, 16
_NW = _NC * _NS
_MPW = _N_MOL // _NW
_IPW = _MPW * _MN

_S_NP = np.zeros((_F, _NN), np.float32)
for _i in range(_F):
    if _i % 4 < 3:
        _S_NP[_i, _i // 4] = 1.0


def _atom_energy_body(n2_ref, s_ref, sw_ref, out_ref):
    x = n2_ref[...]
    sq = x * x
    r2 = jnp.dot(sq, s_ref[...], preferred_element_type=jnp.float32)
    pred = r2 > 1e-6
    r2s = jnp.where(pred, r2, 1.0)
    r6 = r2s * r2s * r2s
    inv6 = 1.0 / r6
    scale = 2.0 * sw_ref[0, 0]
    pe = jnp.where(pred, scale * (inv6 * inv6 - inv6), 0.0)
    ones = jnp.ones((_NN, 1), jnp.float32)
    out_ref[...] = jnp.dot(pe, ones, preferred_element_type=jnp.float32)


def _atom_energies(n2, s_mat, sw):
    return pl.pallas_call(
        _atom_energy_body,
        grid=(_NBLK,),
        in_specs=[
            pl.BlockSpec((_BLK, _F), lambda i: (i, 0)),
            pl.BlockSpec((_F, _NN), lambda i: (0, 0)),
            pl.BlockSpec(memory_space=pltpu.SMEM),
        ],
        out_specs=pl.BlockSpec((_BLK, 1), lambda i: (i, 0)),
        out_shape=jax.ShapeDtypeStruct((_N_ATOMS, 1), jnp.float32),
    )(n2, s_mat, sw)


def _mol_sum_body(idx_hbm, epad_hbm, out_hbm, idx_v, rows_v, out_v, sem):
    wid = lax.axis_index("s") * _NC + lax.axis_index("c")
    pltpu.sync_copy(idx_hbm.at[pl.ds(wid * _IPW, _IPW)], idx_v)
    for s in range(_MN):
        pltpu.async_copy(
            epad_hbm.at[idx_v.at[pl.ds(s * _MPW, _MPW)]],
            rows_v.at[pl.ds(s * _MPW, _MPW)], sem).wait()
    for k in range(_MPW // 16):
        acc = rows_v[pl.ds(k * 16, 16)]
        for s in range(1, _MN):
            acc = acc + rows_v[pl.ds(s * _MPW + k * 16, 16)]
        out_v[pl.ds(k * 16, 16)] = acc
    pltpu.sync_copy(out_v, out_hbm.at[pl.ds(wid * _MPW, _MPW)])


def _mol_sum(idx_t, e_pad):
    mesh = plsc.VectorSubcoreMesh(core_axis_name="c", subcore_axis_name="s")
    fn = functools.partial(
        pl.kernel,
        out_type=jax.ShapeDtypeStruct((_N_MOL,), jnp.float32),
        mesh=mesh,
        scratch_types=[
            pltpu.VMEM((_IPW,), jnp.int32),
            pltpu.VMEM((_IPW,), jnp.float32),
            pltpu.VMEM((_MPW,), jnp.float32),
            pltpu.SemaphoreType.DMA,
        ],
    )(_mol_sum_body)
    return fn(idx_t, e_pad)


def kernel(nlist, positions, box, sample_weight, mol_indices):
    n2 = nlist.reshape(_N_ATOMS, _F)
    sw = jnp.reshape(sample_weight, (1, 1)).astype(jnp.float32)
    e = _atom_energies(n2, jnp.asarray(_S_NP), sw).reshape(_N_ATOMS)

    e_pad = jnp.pad(e, (1, 7))
    idx_t = (mol_indices.reshape(_NW, _MPW, _MN)
             .transpose(0, 2, 1).reshape(_NW * _IPW))
    return _mol_sum(idx_t, e_pad)
